# Initial kernel scaffold; baseline (speedup 1.0000x reference)
import jax
import jax.numpy as jnp
from jax.experimental import pallas as pl


def kernel(x, edge_index, W1, b1, W2, b2):
    raise NotImplementedError("write your pallas kernel here")


# speedup vs baseline: 8.6596x; 8.6596x over previous
import functools

import jax
import jax.numpy as jnp
from jax import lax
from jax.experimental import pallas as pl
from jax.experimental.pallas import tpu as pltpu
from jax.experimental.pallas import tpu_sc as plsc

N = 10000
D = 256
H = 128
E = 160000
NPAD = 10240
EPAD = 163840
CH = 128
TCH = EPAD // 16 // CH
HCH = TCH // 2
DCH = EPAD // 32 // CH
BR = 1024
GRID = NPAD // BR
F32 = jnp.float32

_mesh = plsc.VectorSubcoreMesh(core_axis_name="c", subcore_axis_name="s")


@functools.partial(
    pl.kernel,
    mesh=_mesh,
    out_type=jax.ShapeDtypeStruct((2, NPAD), F32),
    scratch_types=[
        pltpu.VMEM((DCH, CH), jnp.int32),
        pltpu.VMEM((CH,), F32),
        pltpu.VMEM((NPAD // 16,), F32),
        pltpu.VMEM_SHARED((NPAD,), F32),
    ],
)
def _deg_kernel(dst_hbm, out_hbm, didx_v, ones_v, stage_v, hist_sh):
    c = lax.axis_index("c")
    s = lax.axis_index("s")
    wid = s * 2 + c
    seg = NPAD // 16

    for i in range(CH // 16):
        ones_v[pl.ds(i * 16, 16)] = jnp.ones((16,), F32)

    def _zero(i, carry):
        stage_v[pl.ds(i * 16, 16)] = jnp.zeros((16,), F32)
        return carry

    lax.fori_loop(0, seg // 16, _zero, 0)
    pltpu.sync_copy(stage_v, hist_sh.at[pl.ds(s * seg, seg)])
    plsc.subcore_barrier()

    pltpu.sync_copy(dst_hbm.at[wid], didx_v)

    def _accum(j, carry):
        pltpu.sync_copy(ones_v, hist_sh.at[didx_v.at[j]], add=True)
        return carry

    lax.fori_loop(0, DCH, _accum, 0)
    plsc.subcore_barrier()

    pltpu.sync_copy(hist_sh.at[pl.ds(s * seg, seg)], stage_v)
    pltpu.sync_copy(stage_v, out_hbm.at[c, pl.ds(s * seg, seg)])


@functools.partial(
    pl.kernel,
    mesh=_mesh,
    out_type=jax.ShapeDtypeStruct((2, NPAD, H), F32),
    scratch_types=[
        pltpu.VMEM((HCH, CH), jnp.int32),
        pltpu.VMEM((HCH, CH), jnp.int32),
        pltpu.VMEM((CH, H), F32),
        pltpu.VMEM((CH, H), F32),
        pltpu.VMEM_SHARED((NPAD, H), F32),
        pltpu.SemaphoreType.DMA,
        pltpu.SemaphoreType.DMA,
    ],
)
def _agg_kernel(y0, y1, src_hbm, dst_hbm, out_hbm, sidx, didx, ra, rb,
                acc_sh, sem_a, sem_b):
    c = lax.axis_index("c")
    s = lax.axis_index("s")
    seg = NPAD // 16

    def _zrow(r, carry):
        for i in range(H // 16):
            ra[r, pl.ds(i * 16, 16)] = jnp.zeros((16,), F32)
        return carry

    lax.fori_loop(0, CH, _zrow, 0)
    for k in range(seg // CH):
        pltpu.sync_copy(ra, acc_sh.at[pl.ds(s * seg + k * CH, CH)])

    plsc.subcore_barrier()

    def _run(y):
        def start(j, rows, sem):
            pltpu.async_copy(y.at[sidx.at[j]], rows, sem)

        def finish(j, rows, sem):
            pltpu.make_async_copy(y.at[sidx.at[j]], rows, sem).wait()
            pltpu.sync_copy(rows, acc_sh.at[didx.at[j]], add=True)

        for hp in range(2):
            pltpu.sync_copy(src_hbm.at[s, pl.ds(hp * HCH, HCH)], sidx)
            pltpu.sync_copy(dst_hbm.at[s, pl.ds(hp * HCH, HCH)], didx)
            start(0, ra, sem_a)

            def body(t, carry):
                j = 2 * t
                start(j + 1, rb, sem_b)
                finish(j, ra, sem_a)

                @pl.when(j + 2 < HCH)
                def _():
                    start(j + 2, ra, sem_a)

                finish(j + 1, rb, sem_b)
                return carry

            lax.fori_loop(0, HCH // 2, body, 0)

    @pl.when(c == 0)
    def _():
        _run(y0)

    @pl.when(c == 1)
    def _():
        _run(y1)

    plsc.subcore_barrier()
    for k in range(seg // CH):
        pltpu.sync_copy(acc_sh.at[pl.ds(s * seg + k * CH, CH)], ra)
        pltpu.sync_copy(ra, out_hbm.at[c, pl.ds(s * seg + k * CH, CH)])


def _dis_from_hist(hist_ref):
    deg = hist_ref[0, :] + hist_ref[1, :] + 1.0
    return lax.rsqrt(deg)[:, None]


def _tc1_body(hist_ref, x_ref, w1_ref, y0_ref, y1_ref):
    dis = _dis_from_hist(hist_ref)
    xw = jnp.dot(x_ref[...], w1_ref[...], preferred_element_type=F32)
    y = xw * dis
    y0_ref[...] = y[:, :H]
    y1_ref[...] = y[:, H:]


def _tc2_body(hist_ref, acc_ref, y0_ref, y1_ref, b1_ref, w2_ref, z0_ref, z1_ref):
    dis = _dis_from_hist(hist_ref)
    h0 = jnp.maximum((acc_ref[0] + y0_ref[...]) * dis + b1_ref[0, :H][None, :], 0.0)
    h1 = jnp.maximum((acc_ref[1] + y1_ref[...]) * dis + b1_ref[0, H:][None, :], 0.0)
    z = (jnp.dot(h0, w2_ref[:H, :], preferred_element_type=F32)
         + jnp.dot(h1, w2_ref[H:, :], preferred_element_type=F32)) * dis
    z0_ref[...] = z[:, :H]
    z1_ref[...] = z[:, H:]


def _tc3_body(hist_ref, acc_ref, z0_ref, z1_ref, b2_ref, o_ref):
    dis = _dis_from_hist(hist_ref)
    o_ref[:, :H] = (acc_ref[0] + z0_ref[...]) * dis + b2_ref[0, :H][None, :]
    o_ref[:, H:] = (acc_ref[1] + z1_ref[...]) * dis + b2_ref[0, H:][None, :]


_hist_spec = pl.BlockSpec((2, BR), lambda i: (0, i))
_row_spec = pl.BlockSpec((BR, D), lambda i: (i, 0))
_half_spec = pl.BlockSpec((BR, H), lambda i: (i, 0))
_acc_spec = pl.BlockSpec((2, BR, H), lambda i: (0, i, 0))
_w_spec = pl.BlockSpec((D, D), lambda i: (0, 0))
_b_spec = pl.BlockSpec((1, D), lambda i: (0, 0))

_tc1 = pl.pallas_call(
    _tc1_body,
    grid=(GRID,),
    in_specs=[_hist_spec, _row_spec, _w_spec],
    out_specs=[_half_spec, _half_spec],
    out_shape=[jax.ShapeDtypeStruct((NPAD, H), F32)] * 2,
)

_tc2 = pl.pallas_call(
    _tc2_body,
    grid=(GRID,),
    in_specs=[_hist_spec, _acc_spec, _half_spec, _half_spec, _b_spec, _w_spec],
    out_specs=[_half_spec, _half_spec],
    out_shape=[jax.ShapeDtypeStruct((NPAD, H), F32)] * 2,
)

_tc3 = pl.pallas_call(
    _tc3_body,
    grid=(GRID,),
    in_specs=[_hist_spec, _acc_spec, _half_spec, _half_spec, _b_spec],
    out_specs=_row_spec,
    out_shape=jax.ShapeDtypeStruct((NPAD, D), F32),
)


def kernel(x, edge_index, W1, b1, W2, b2):
    src = edge_index[0].astype(jnp.int32)
    dst = edge_index[1].astype(jnp.int32)
    padn = EPAD - E
    src_p = jnp.concatenate([src, jnp.zeros((padn,), jnp.int32)])
    dst_p = jnp.concatenate([dst, jnp.full((padn,), N, jnp.int32)])
    src16 = src_p.reshape(16, TCH, CH)
    dst16 = dst_p.reshape(16, TCH, CH)
    dst32 = dst_p.reshape(32, DCH, CH)
    xp = jnp.pad(x, ((0, NPAD - N), (0, 0)))
    b1r = b1.reshape(1, D)
    b2r = b2.reshape(1, D)

    hist = _deg_kernel(dst32)
    y0, y1 = _tc1(hist, xp, W1)
    acc1 = _agg_kernel(y0, y1, src16, dst16)
    z0, z1 = _tc2(hist, acc1, y0, y1, b1r, W2)
    acc2 = _agg_kernel(z0, z1, src16, dst16)
    out = _tc3(hist, acc2, z0, z1, b2r)
    return out[:N]
